# baseline (speedup 1.0000x reference)
import jax
import jax.numpy as jnp
from jax.experimental import pallas as pl
from jax.experimental.pallas import tpu as pltpu
from jax.experimental.pallas import tpu_sc as plsc


_GATHER_WINDOW = 128
_SUBROW = 128


def _sc_gather(table, idx_row):
    M = idx_row.shape[1]
    W = table.shape[1]
    mesh = plsc.VectorSubcoreMesh(core_axis_name="c", subcore_axis_name="s")

    @pl.kernel(
        out_type=jax.ShapeDtypeStruct((M, W), table.dtype),
        mesh=mesh,
    )
    def gather_kernel(tab_hbm, idx_hbm, out_hbm):
        def body(i_vmem, o_vmem):
            pltpu.sync_copy(tab_hbm.at[i_vmem.at[0]], o_vmem)

        pltpu.emit_pipeline(
            body,
            grid=(M // _GATHER_WINDOW,),
            in_specs=[pl.BlockSpec((1, _GATHER_WINDOW), lambda i: (0, i))],
            out_specs=[pl.BlockSpec((_GATHER_WINDOW, W), lambda i: (i, 0))],
            core_axis_name=("c", "s"),
            dimension_semantics=(pltpu.PARALLEL,),
        )(idx_hbm, out_hbm)

    return gather_kernel(table, idx_row)


def _ln_body(tok_ref, pos_ref, g_ref, b_ref, o_ref):
    x = tok_ref[...] + pos_ref[...]
    mean = jnp.mean(x, axis=-1, keepdims=True)
    cent = x - mean
    var = jnp.mean(cent * cent, axis=-1, keepdims=True)
    y = cent * jax.lax.rsqrt(var + 1e-5) * g_ref[...] + b_ref[...]
    o_ref[...] = y.astype(jnp.bfloat16)


def _ln(tok_emb, pos_emb, gamma, beta):
    T, D = tok_emb.shape
    ROWS = min(256, T)
    return pl.pallas_call(
        _ln_body,
        grid=(T // ROWS,),
        in_specs=[
            pl.BlockSpec((ROWS, D), lambda i: (i, 0)),
            pl.BlockSpec((ROWS, D), lambda i: (i, 0)),
            pl.BlockSpec((1, D), lambda i: (0, 0)),
            pl.BlockSpec((1, D), lambda i: (0, 0)),
        ],
        out_specs=pl.BlockSpec((ROWS, D), lambda i: (i, 0)),
        out_shape=jax.ShapeDtypeStruct((T, D), jnp.bfloat16),
    )(tok_emb, pos_emb, gamma.reshape(1, D), beta.reshape(1, D))


def _mm_body(x_ref, w_ref, o_ref):
    w = w_ref[...].astype(jnp.bfloat16)
    o_ref[...] = jax.lax.dot_general(
        x_ref[...], w, (((1,), (1,)), ((), ())),
        preferred_element_type=jnp.float32,
    )


def _lm_head(x_bf16, W_lm):
    T, D = x_bf16.shape
    V = W_lm.shape[0]
    VT = 512
    return pl.pallas_call(
        _mm_body,
        grid=(pl.cdiv(V, VT),),
        in_specs=[
            pl.BlockSpec((T, D), lambda j: (0, 0)),
            pl.BlockSpec((VT, D), lambda j: (j, 0)),
        ],
        out_specs=pl.BlockSpec((T, VT), lambda j: (0, j)),
        out_shape=jax.ShapeDtypeStruct((T, V), jnp.float32),
    )(x_bf16, W_lm)


def kernel(idx, tok_table, pos_table, ln_gamma, ln_beta, W_lm):
    B, T = idx.shape
    D = tok_table.shape[1]
    V = W_lm.shape[0]
    n_sub = D // _SUBROW
    idx_exp = (idx.reshape(-1, 1).astype(jnp.int32) * n_sub
               + jnp.arange(n_sub, dtype=jnp.int32).reshape(1, -1))
    idx_row = idx_exp.reshape(1, B * T * n_sub)
    table = tok_table.reshape(tok_table.shape[0] * n_sub, _SUBROW)
    gathered = _sc_gather(table, idx_row)
    tok_emb = gathered.reshape(B * T, D)
    x = _ln(tok_emb, pos_table[:T], ln_gamma, ln_beta)
    logits = _lm_head(x, W_lm)
    return logits.reshape(B, T, V)


# speedup vs baseline: 1.5654x; 1.5654x over previous
import jax
import jax.numpy as jnp
from jax.experimental import pallas as pl
from jax.experimental.pallas import tpu as pltpu
from jax.experimental.pallas import tpu_sc as plsc


def _sc_gather(table, idx_flat):
    B = idx_flat.shape[0]
    D = table.shape[1]
    NW = 32
    b_per_w = B // NW
    chunk = b_per_w
    while chunk * D * 4 > 256 * 1024:
        chunk //= 2
    n_chunks = b_per_w // chunk
    mesh = plsc.VectorSubcoreMesh(core_axis_name="c", subcore_axis_name="s")

    @pl.kernel(
        out_type=jax.ShapeDtypeStruct((B, D), table.dtype),
        mesh=mesh,
        scratch_types=[
            pltpu.VMEM((chunk,), jnp.int32),
            pltpu.VMEM((chunk, D), table.dtype),
            pltpu.SemaphoreType.DMA,
        ],
    )
    def gather_kernel(tab_hbm, idx_hbm, out_hbm, idx_v, rows_v, sem):
        wid = jax.lax.axis_index("s") * 2 + jax.lax.axis_index("c")
        base = wid * b_per_w

        @pl.loop(0, n_chunks)
        def _(ci):
            off = base + ci * chunk
            pltpu.sync_copy(idx_hbm.at[pl.ds(off, chunk)], idx_v)
            pltpu.async_copy(tab_hbm.at[idx_v], rows_v, sem).wait()
            pltpu.sync_copy(rows_v, out_hbm.at[pl.ds(off, chunk)])

    return gather_kernel(table, idx_flat)


def _ln_body(tok_ref, pos_ref, g_ref, b_ref, o_ref):
    x = tok_ref[...] + pos_ref[...]
    mean = jnp.mean(x, axis=-1, keepdims=True)
    cent = x - mean
    var = jnp.mean(cent * cent, axis=-1, keepdims=True)
    y = cent * jax.lax.rsqrt(var + 1e-5) * g_ref[...] + b_ref[...]
    o_ref[...] = y.astype(jnp.bfloat16)


def _ln(tok_emb, pos_emb, gamma, beta):
    T, D = tok_emb.shape
    ROWS = min(256, T)
    return pl.pallas_call(
        _ln_body,
        grid=(T // ROWS,),
        in_specs=[
            pl.BlockSpec((ROWS, D), lambda i: (i, 0)),
            pl.BlockSpec((ROWS, D), lambda i: (i, 0)),
            pl.BlockSpec((1, D), lambda i: (0, 0)),
            pl.BlockSpec((1, D), lambda i: (0, 0)),
        ],
        out_specs=pl.BlockSpec((ROWS, D), lambda i: (i, 0)),
        out_shape=jax.ShapeDtypeStruct((T, D), jnp.bfloat16),
    )(tok_emb, pos_emb, gamma.reshape(1, D), beta.reshape(1, D))


def _mm_body(x_ref, w_ref, o_ref):
    w = w_ref[...].astype(jnp.bfloat16)
    o_ref[...] = jax.lax.dot_general(
        x_ref[...], w, (((1,), (1,)), ((), ())),
        preferred_element_type=jnp.float32,
    )


def _lm_head(x_bf16, W_lm):
    T, D = x_bf16.shape
    V = W_lm.shape[0]
    VT = 512
    return pl.pallas_call(
        _mm_body,
        grid=(pl.cdiv(V, VT),),
        in_specs=[
            pl.BlockSpec((T, D), lambda j: (0, 0)),
            pl.BlockSpec((VT, D), lambda j: (j, 0)),
        ],
        out_specs=pl.BlockSpec((T, VT), lambda j: (0, j)),
        out_shape=jax.ShapeDtypeStruct((T, V), jnp.float32),
    )(x_bf16, W_lm)


def kernel(idx, tok_table, pos_table, ln_gamma, ln_beta, W_lm):
    B, T = idx.shape
    D = tok_table.shape[1]
    V = W_lm.shape[0]
    idx_flat = idx.reshape(B * T).astype(jnp.int32)
    tok_emb = _sc_gather(tok_table, idx_flat)
    x = _ln(tok_emb, pos_table[:T], ln_gamma, ln_beta)
    logits = _lm_head(x, W_lm)
    return logits.reshape(B, T, V)


# speedup vs baseline: 2.2160x; 1.4156x over previous
import jax
import jax.numpy as jnp
from jax.experimental import pallas as pl
from jax.experimental.pallas import tpu as pltpu
from jax.experimental.pallas import tpu_sc as plsc


def _sc_gather(table, idx_flat):
    B = idx_flat.shape[0]
    D = table.shape[1]
    NW = 32
    b_per_w = B // NW
    chunk = b_per_w
    while chunk * D * 4 > 256 * 1024:
        chunk //= 2
    n_chunks = b_per_w // chunk
    mesh = plsc.VectorSubcoreMesh(core_axis_name="c", subcore_axis_name="s")

    @pl.kernel(
        out_type=jax.ShapeDtypeStruct((B, D), table.dtype),
        mesh=mesh,
        scratch_types=[
            pltpu.VMEM((chunk,), jnp.int32),
            pltpu.VMEM((chunk, D), table.dtype),
            pltpu.SemaphoreType.DMA,
        ],
    )
    def gather_kernel(tab_hbm, idx_hbm, out_hbm, idx_v, rows_v, sem):
        wid = jax.lax.axis_index("s") * 2 + jax.lax.axis_index("c")
        base = wid * b_per_w

        @pl.loop(0, n_chunks)
        def _(ci):
            off = base + ci * chunk
            pltpu.sync_copy(idx_hbm.at[pl.ds(off, chunk)], idx_v)
            pltpu.async_copy(tab_hbm.at[idx_v], rows_v, sem).wait()
            pltpu.sync_copy(rows_v, out_hbm.at[pl.ds(off, chunk)])

    return gather_kernel(table, idx_flat)


def _ln_body(tok_ref, pos_ref, g_ref, b_ref, o_ref):
    x = tok_ref[...] + pos_ref[...]
    mean = jnp.mean(x, axis=-1, keepdims=True)
    cent = x - mean
    var = jnp.mean(cent * cent, axis=-1, keepdims=True)
    y = cent * jax.lax.rsqrt(var + 1e-5) * g_ref[...] + b_ref[...]
    o_ref[...] = y.astype(jnp.bfloat16)


def _ln(tok_emb, pos_emb, gamma, beta):
    T, D = tok_emb.shape
    ROWS = min(256, T)
    return pl.pallas_call(
        _ln_body,
        grid=(T // ROWS,),
        in_specs=[
            pl.BlockSpec((ROWS, D), lambda i: (i, 0)),
            pl.BlockSpec((ROWS, D), lambda i: (i, 0)),
            pl.BlockSpec((1, D), lambda i: (0, 0)),
            pl.BlockSpec((1, D), lambda i: (0, 0)),
        ],
        out_specs=pl.BlockSpec((ROWS, D), lambda i: (i, 0)),
        out_shape=jax.ShapeDtypeStruct((T, D), jnp.bfloat16),
    )(tok_emb, pos_emb, gamma.reshape(1, D), beta.reshape(1, D))


def _mm_body(w_ref, x_ref, o_ref):
    w = w_ref[...].astype(jnp.bfloat16)
    acc = jax.lax.dot_general(
        w, x_ref[...], (((1,), (1,)), ((), ())),
        preferred_element_type=jnp.float32,
    )
    vt, t = acc.shape
    o_ref[...] = acc.reshape(vt * (t // 128), 128)


def _lm_head(x_bf16, W_lm):
    T, D = x_bf16.shape
    V = W_lm.shape[0]
    VT = 512
    NT = T // 128
    return pl.pallas_call(
        _mm_body,
        grid=(pl.cdiv(V, VT),),
        in_specs=[
            pl.BlockSpec((VT, D), lambda j: (j, 0)),
            pl.BlockSpec((T, D), lambda j: (0, 0)),
        ],
        out_specs=pl.BlockSpec((VT * NT, 128), lambda j: (j, 0)),
        out_shape=jax.ShapeDtypeStruct((V * NT, 128), jnp.float32),
    )(W_lm, x_bf16)


def kernel(idx, tok_table, pos_table, ln_gamma, ln_beta, W_lm):
    B, T = idx.shape
    D = tok_table.shape[1]
    V = W_lm.shape[0]
    idx_flat = idx.reshape(B * T).astype(jnp.int32)
    tok_emb = _sc_gather(tok_table, idx_flat)
    x = _ln(tok_emb, pos_table[:T], ln_gamma, ln_beta)
    logits_t = _lm_head(x, W_lm)
    nt = T // 128
    return logits_t.reshape(V, nt, 128).transpose(1, 2, 0).reshape(B, T, V)


# speedup vs baseline: 2.2803x; 1.0290x over previous
import jax
import jax.numpy as jnp
from jax.experimental import pallas as pl
from jax.experimental.pallas import tpu as pltpu
from jax.experimental.pallas import tpu_sc as plsc


def _sc_gather(table, idx_flat):
    B = idx_flat.shape[0]
    D = table.shape[1]
    NW = 32
    b_per_w = B // NW
    chunk = b_per_w
    while chunk * D * 4 > 256 * 1024:
        chunk //= 2
    n_chunks = b_per_w // chunk
    mesh = plsc.VectorSubcoreMesh(core_axis_name="c", subcore_axis_name="s")

    @pl.kernel(
        out_type=jax.ShapeDtypeStruct((B, D), table.dtype),
        mesh=mesh,
        scratch_types=[
            pltpu.VMEM((chunk,), jnp.int32),
            pltpu.VMEM((chunk, D), table.dtype),
            pltpu.SemaphoreType.DMA,
        ],
    )
    def gather_kernel(tab_hbm, idx_hbm, out_hbm, idx_v, rows_v, sem):
        wid = jax.lax.axis_index("s") * 2 + jax.lax.axis_index("c")
        base = wid * b_per_w

        @pl.loop(0, n_chunks)
        def _(ci):
            off = base + ci * chunk
            pltpu.sync_copy(idx_hbm.at[pl.ds(off, chunk)], idx_v)
            pltpu.async_copy(tab_hbm.at[idx_v], rows_v, sem).wait()
            pltpu.sync_copy(rows_v, out_hbm.at[pl.ds(off, chunk)])

    return gather_kernel(table, idx_flat)


def _ln_body(tok_ref, pos_ref, g_ref, b_ref, o_ref):
    x = tok_ref[...] + pos_ref[...]
    mean = jnp.mean(x, axis=-1, keepdims=True)
    cent = x - mean
    var = jnp.mean(cent * cent, axis=-1, keepdims=True)
    y = cent * jax.lax.rsqrt(var + 1e-5) * g_ref[...] + b_ref[...]
    o_ref[...] = y.astype(jnp.bfloat16)


def _ln(tok_emb, pos_emb, gamma, beta):
    T, D = tok_emb.shape
    ROWS = min(256, T)
    return pl.pallas_call(
        _ln_body,
        grid=(T // ROWS,),
        in_specs=[
            pl.BlockSpec((ROWS, D), lambda i: (i, 0)),
            pl.BlockSpec((ROWS, D), lambda i: (i, 0)),
            pl.BlockSpec((1, D), lambda i: (0, 0)),
            pl.BlockSpec((1, D), lambda i: (0, 0)),
        ],
        out_specs=pl.BlockSpec((ROWS, D), lambda i: (i, 0)),
        out_shape=jax.ShapeDtypeStruct((T, D), jnp.bfloat16),
    )(tok_emb, pos_emb, gamma.reshape(1, D), beta.reshape(1, D))


def _mm_body(w_ref, x_ref, o_ref):
    w = w_ref[...].astype(jnp.bfloat16)
    acc = jax.lax.dot_general(
        w, x_ref[...], (((1,), (1,)), ((), ())),
        preferred_element_type=jnp.float32,
    )
    vt, t = acc.shape
    o_ref[...] = acc.reshape(vt * (t // 128), 128)


def _lm_head(x_bf16, W_lm):
    T, D = x_bf16.shape
    V = W_lm.shape[0]
    VT = 1024
    NT = T // 128
    return pl.pallas_call(
        _mm_body,
        grid=(pl.cdiv(V, VT),),
        in_specs=[
            pl.BlockSpec((VT, D), lambda j: (j, 0)),
            pl.BlockSpec((T, D), lambda j: (0, 0)),
        ],
        out_specs=pl.BlockSpec((VT * NT, 128), lambda j: (j, 0)),
        out_shape=jax.ShapeDtypeStruct((V * NT, 128), jnp.float32),
    )(W_lm, x_bf16)


def kernel(idx, tok_table, pos_table, ln_gamma, ln_beta, W_lm):
    B, T = idx.shape
    D = tok_table.shape[1]
    V = W_lm.shape[0]
    idx_flat = idx.reshape(B * T).astype(jnp.int32)
    tok_emb = _sc_gather(tok_table, idx_flat)
    x = _ln(tok_emb, pos_table[:T], ln_gamma, ln_beta)
    logits_t = _lm_head(x, W_lm)
    nt = T // 128
    return logits_t.reshape(V, nt, 128).transpose(1, 2, 0).reshape(B, T, V)
